# baseline (speedup 1.0000x reference)
import functools

import jax
import jax.numpy as jnp
from jax import lax
from jax.experimental import pallas as pl
from jax.experimental.pallas import tpu as pltpu
from jax.experimental.pallas import tpu_sc as plsc

BATCH, SEQ, D = 1024, 512, 128
D4 = 32
NW = 32
BPW = BATCH // NW
CHUNK = 128
NCHUNK = BPW * SEQ // CHUNK
NBUF = 4


def _sc_body(x_hbm, w_hbm, h_hbm, pos_hbm, wk_hbm, hr_hbm, dec_hbm, out_hbm,
             pos_v, xs, wk_v, hr_v, wbufs, hbufs, dec_v, isems, osems):
    wid = lax.axis_index("s") * 2 + lax.axis_index("c")

    pltpu.sync_copy(wk_hbm, wk_v)
    pltpu.sync_copy(hr_hbm, hr_v)
    pltpu.sync_copy(pos_hbm, pos_v)
    pltpu.sync_copy(dec_hbm, dec_v)
    decvec = dec_v[...]

    def scale_row(r, carry):
        r_f = r.astype(jnp.float32)
        dw = jnp.exp(decvec * ((r_f - (SEQ - 1)) * (1.0 / SEQ)))
        for c in range(4):
            sl = pl.ds(r * 64 + c * 16, 16)
            pos_v[sl] = pos_v[sl] * dw
        return carry

    lax.fori_loop(0, SEQ, scale_row, 0)

    base_pt = wid * (BPW * SEQ)

    def start_in(k, b):
        pt0 = base_pt + k * CHUNK
        pltpu.async_copy(x_hbm.at[pl.ds(pt0 * D, CHUNK * D)], xs[b], isems[b])
        pltpu.async_copy(w_hbm.at[pl.ds(pt0, CHUNK)], wbufs[b], isems[b])
        pltpu.async_copy(h_hbm.at[pl.ds(pt0, CHUNK)], hbufs[b], isems[b])

    def wait_in(k, b):
        pt0 = base_pt + k * CHUNK
        pltpu.make_async_copy(x_hbm.at[pl.ds(pt0 * D, CHUNK * D)], xs[b],
                              isems[b]).wait()
        pltpu.make_async_copy(w_hbm.at[pl.ds(pt0, CHUNK)], wbufs[b],
                              isems[b]).wait()
        pltpu.make_async_copy(h_hbm.at[pl.ds(pt0, CHUNK)], hbufs[b],
                              isems[b]).wait()

    def start_out(k, b):
        pt0 = base_pt + k * CHUNK
        pltpu.async_copy(xs[b], out_hbm.at[pl.ds(pt0 * D, CHUNK * D)],
                         osems[b])

    def wait_out(k, b):
        pt0 = base_pt + k * CHUNK
        pltpu.make_async_copy(xs[b], out_hbm.at[pl.ds(pt0 * D, CHUNK * D)],
                              osems[b]).wait()

    def compute(k, b):
        xb = xs[b]
        s_base = lax.rem(k * CHUNK, SEQ)
        lanes = lax.iota(jnp.int32, 16)

        def group_body(g, c2):
            p0 = g * 16
            wvec = wbufs[b][pl.ds(p0, 16)] * D4
            hvec = hbufs[b][pl.ds(p0, 16)] * D4
            xbase = lanes * D + p0 * D
            for j in range(D4):
                vw = plsc.load_gather(wk_v, [wvec + j])
                plsc.addupdate_scatter(xb, [xbase + (64 + j)], vw)
                vh = plsc.load_gather(hr_v, [hvec + j])
                plsc.addupdate_scatter(xb, [xbase + (96 + j)], vh)
            for l in range(16):
                xo = (p0 + l) * D
                so = (s_base + p0 + l) * 64
                for c in range(4):
                    plsc.addupdate(xb.at[pl.ds(xo + c * 16, 16)],
                                   pos_v[pl.ds(so + c * 16, 16)])
            return c2

        lax.fori_loop(0, CHUNK // 16, group_body, 0)

    start_in(0, 0)
    start_in(1, 1)

    def ring_iter(i, carry):
        for bb in range(NBUF):
            k = i * NBUF + bb
            b = bb
            wait_in(k, b)
            compute(k, b)
            start_out(k, b)
            b2 = (bb + 2) % NBUF

            @pl.when(k >= 2)
            def _():
                wait_out(k - 2, b2)

            @pl.when(k + 2 < NCHUNK)
            def _():
                start_in(k + 2, b2)

        return carry

    lax.fori_loop(0, NCHUNK // NBUF, ring_iter, 0)
    wait_out(NCHUNK - 2, (NCHUNK - 2) % NBUF)
    wait_out(NCHUNK - 1, (NCHUNK - 1) % NBUF)


@jax.jit
def kernel(x, weekdays, hours, pos_table, weekday_table, hour_table, decay):
    mesh = plsc.VectorSubcoreMesh(core_axis_name="c", subcore_axis_name="s")
    dec16 = jnp.full((16,), decay[0], jnp.float32)
    run = pl.kernel(
        _sc_body,
        out_type=jax.ShapeDtypeStruct((BATCH * SEQ * D,), jnp.float32),
        mesh=mesh,
        compiler_params=pltpu.CompilerParams(needs_layout_passes=False),
        scratch_types=[
            pltpu.VMEM((SEQ * 64,), jnp.float32),
            [pltpu.VMEM((CHUNK * D,), jnp.float32)] * NBUF,
            pltpu.VMEM((7 * D4,), jnp.float32),
            pltpu.VMEM((24 * D4,), jnp.float32),
            [pltpu.VMEM((CHUNK,), jnp.int32)] * NBUF,
            [pltpu.VMEM((CHUNK,), jnp.int32)] * NBUF,
            pltpu.VMEM((16,), jnp.float32),
            [pltpu.SemaphoreType.DMA] * NBUF,
            [pltpu.SemaphoreType.DMA] * NBUF,
        ],
    )
    out = run(x.reshape(-1), weekdays.reshape(-1), hours.reshape(-1),
              pos_table.reshape(-1), weekday_table.reshape(-1),
              hour_table.reshape(-1), dec16)
    return out.reshape(BATCH, SEQ, D)


# speedup vs baseline: 4.5989x; 4.5989x over previous
import functools

import jax
import jax.numpy as jnp
from jax import lax
from jax.experimental import pallas as pl
from jax.experimental.pallas import tpu as pltpu
from jax.experimental.pallas import tpu_sc as plsc

BATCH, SEQ, D = 1024, 512, 128
D4 = 32
NW = 32
BPW = BATCH // NW
CHUNK = 128
NCHUNK = BPW * SEQ // CHUNK
NBUF = 4


def _sc_body(x_hbm, w_hbm, h_hbm, pos_hbm, wk_hbm, hr_hbm, dec_hbm, out_hbm,
             pos_v, xs, wk_v, hr_v, wvb, hvb, dec_v, isems, osems):
    wid = lax.axis_index("s") * 2 + lax.axis_index("c")

    pltpu.sync_copy(wk_hbm, wk_v)
    pltpu.sync_copy(hr_hbm, hr_v)
    pltpu.sync_copy(pos_hbm, pos_v)
    pltpu.sync_copy(dec_hbm, dec_v)
    decvec = dec_v[...]

    def scale_row(r, carry):
        r_f = r.astype(jnp.float32)
        dw = jnp.exp(decvec * ((r_f - (SEQ - 1)) * (1.0 / SEQ)))
        for c in range(4):
            sl = pl.ds(r * 64 + c * 16, 16)
            pos_v[sl] = pos_v[sl] * dw
        return carry

    lax.fori_loop(0, SEQ, scale_row, 0)

    base_pt = wid * (BPW * SEQ)

    def start_in(k, b):
        pt0 = base_pt + k * CHUNK
        pltpu.async_copy(x_hbm.at[pl.ds(pt0 * D, CHUNK * D)], xs[b], isems[b])
        pltpu.async_copy(w_hbm.at[pl.ds(pt0, CHUNK)], wvb[b], isems[b])
        pltpu.async_copy(h_hbm.at[pl.ds(pt0, CHUNK)], hvb[b], isems[b])

    def wait_in(k, b):
        pt0 = base_pt + k * CHUNK
        pltpu.make_async_copy(x_hbm.at[pl.ds(pt0 * D, CHUNK * D)], xs[b],
                              isems[b]).wait()
        pltpu.make_async_copy(w_hbm.at[pl.ds(pt0, CHUNK)], wvb[b],
                              isems[b]).wait()
        pltpu.make_async_copy(h_hbm.at[pl.ds(pt0, CHUNK)], hvb[b],
                              isems[b]).wait()

    def start_out(k, b):
        pt0 = base_pt + k * CHUNK
        pltpu.async_copy(xs[b], out_hbm.at[pl.ds(pt0 * D, CHUNK * D)],
                         osems[b])

    def wait_out(k, b):
        pt0 = base_pt + k * CHUNK
        pltpu.make_async_copy(xs[b], out_hbm.at[pl.ds(pt0 * D, CHUNK * D)],
                              osems[b]).wait()

    def compute(k, b):
        xb = xs[b]
        s_base = lax.rem(k * CHUNK, SEQ)

        @plsc.parallel_loop(0, CHUNK // 16, 1, carry=jnp.int32(0))
        def group_body(g, c2):
            p0 = g * 16
            wvec = wvb[b][pl.ds(p0, 16)] * D4
            hvec = hvb[b][pl.ds(p0, 16)] * D4
            for l in range(16):
                xo = (p0 + l) * D
                so = (s_base + p0 + l) * 64
                wrow = wvec[l]
                hrow = hvec[l]
                for c in range(4):
                    plsc.addupdate(xb.at[pl.ds(xo + c * 16, 16)],
                                   pos_v[pl.ds(so + c * 16, 16)])
                for q in range(2):
                    plsc.addupdate(xb.at[pl.ds(xo + 64 + q * 16, 16)],
                                   wk_v[pl.ds(wrow + q * 16, 16)])
                    plsc.addupdate(xb.at[pl.ds(xo + 96 + q * 16, 16)],
                                   hr_v[pl.ds(hrow + q * 16, 16)])
            return c2

    start_in(0, 0)
    start_in(1, 1)

    def ring_iter(i, carry):
        for bb in range(NBUF):
            k = i * NBUF + bb
            b = bb
            wait_in(k, b)
            compute(k, b)
            start_out(k, b)
            b2 = (bb + 2) % NBUF

            @pl.when(k >= 2)
            def _():
                wait_out(k - 2, b2)

            @pl.when(k + 2 < NCHUNK)
            def _():
                start_in(k + 2, b2)

        return carry

    lax.fori_loop(0, NCHUNK // NBUF, ring_iter, 0)
    wait_out(NCHUNK - 2, (NCHUNK - 2) % NBUF)
    wait_out(NCHUNK - 1, (NCHUNK - 1) % NBUF)


@jax.jit
def kernel(x, weekdays, hours, pos_table, weekday_table, hour_table, decay):
    mesh = plsc.VectorSubcoreMesh(core_axis_name="c", subcore_axis_name="s")
    dec16 = jnp.full((16,), decay[0], jnp.float32)
    run = pl.kernel(
        _sc_body,
        out_type=jax.ShapeDtypeStruct((BATCH * SEQ * D,), jnp.float32),
        mesh=mesh,
        compiler_params=pltpu.CompilerParams(needs_layout_passes=False),
        scratch_types=[
            pltpu.VMEM((SEQ * 64,), jnp.float32),
            [pltpu.VMEM((CHUNK * D,), jnp.float32)] * NBUF,
            pltpu.VMEM((7 * D4,), jnp.float32),
            pltpu.VMEM((24 * D4,), jnp.float32),
            [pltpu.VMEM((CHUNK,), jnp.int32)] * NBUF,
            [pltpu.VMEM((CHUNK,), jnp.int32)] * NBUF,
            pltpu.VMEM((16,), jnp.float32),
            [pltpu.SemaphoreType.DMA] * NBUF,
            [pltpu.SemaphoreType.DMA] * NBUF,
        ],
    )
    out = run(x.reshape(-1), weekdays.reshape(-1), hours.reshape(-1),
              pos_table.reshape(-1), weekday_table.reshape(-1),
              hour_table.reshape(-1), dec16)
    return out.reshape(BATCH, SEQ, D)
